# baseline (speedup 1.0000x reference)
import functools

import jax
import jax.numpy as jnp
from jax.experimental import pallas as pl
from jax.experimental.pallas import tpu as pltpu
from jax.experimental.pallas import tpu_sc as plsc

N = 4096
C = 768
K_TOPK = 32
NUM_SEL = 1229
BLK = 256

NW = 32
ROWS_PER_W = N // NW
NVREG = N // 16
HB = 256
RANGE = 3.02
W1 = RANGE / HB
W2 = W1 / HB
W3 = W2 / HB
S1 = HB / RANGE
S2 = HB / W1
S3 = HB / W2

_CONST_CACHE = {}


def _noise():
    if "noise" not in _CONST_CACHE:
        _CONST_CACHE["noise"] = (
            jax.random.uniform(jax.random.key(42), (N, N), dtype=jnp.float32) * 0.01
        )
    return _CONST_CACHE["noise"]


def _anomaly_vec(inputs_init, outputs_init):
    gap_list = jnp.mean(jnp.mean(jnp.abs(inputs_init - outputs_init), axis=1), axis=0)
    gap_list_ = jax.nn.sigmoid(jax.lax.stop_gradient(gap_list))
    neg_vals, small_idx = jax.lax.top_k(-gap_list_, NUM_SEL)
    topk_asc = -neg_vals
    topk_ = topk_asc[::-1]
    topk_idx = small_idx[::-1]
    threshold = jnp.mean(topk_)
    valid = topk_ > threshold
    anomaly_vals = jnp.where(valid, topk_, jnp.zeros_like(topk_))
    return jnp.zeros((N,), jnp.float32).at[topk_idx].set(anomaly_vals)


def _stats_body(z_ref, ms_ref):
    z = z_ref[...]
    g = jax.lax.dot_general(
        z, z, (((0,), (0,)), ((), ())), preferred_element_type=jnp.float32
    )
    sumsq = jnp.sum(g * g)
    s = jnp.sum(z, axis=0, keepdims=True)
    total = jnp.sum(s * s)
    n2 = float(N) * float(N)
    mean = total / n2
    var = (sumsq - total * (total / n2)) / (n2 - 1.0)
    ms_ref[0] = mean
    ms_ref[1] = jnp.sqrt(var)


def _tc_body(z_ref, zt_ref, arow_ref, acol_ref, ms_ref, out_ref):
    i = pl.program_id(0)
    base = i * BLK
    x = jax.lax.dot_general(
        z_ref[...], zt_ref[...], (((1,), (0,)), ((), ())),
        preferred_element_type=jnp.float32,
    )
    mean = ms_ref[0]
    std = ms_ref[1]
    adjb = jax.nn.sigmoid((x - mean) / (std + 1e-8))
    arow = arow_ref[...]
    acol = acol_ref[...]
    adjb = adjb + arow + acol
    col_ids = jax.lax.broadcasted_iota(jnp.int32, (BLK, N), 1)
    row_ids = jax.lax.broadcasted_iota(jnp.int32, (BLK, N), 0) + base
    out_ref[...] = adjb - jnp.where(col_ids == row_ids, arow, 0.0)


def _sc_scan_hist(hist, above):
    gs = [hist[pl.ds(g * 16, 16)] for g in range(16)]
    sums = [jnp.sum(g) for g in gs]
    suf = [None] * 17
    acc = above
    suf[16] = above
    for g in range(15, -1, -1):
        acc = acc + sums[g]
        suf[g] = acc
    k = jnp.int32(K_TOPK)
    zero16 = jnp.zeros((16,), jnp.int32)
    gsel = jnp.int32(0)
    hv = zero16
    above_g = jnp.int32(0)
    for g in range(16):
        cond = (suf[g] >= k) & (suf[g + 1] < k)
        gsel = gsel + jnp.where(cond, jnp.int32(g), jnp.int32(0))
        hv = hv + jnp.where(cond, gs[g], zero16)
        above_g = above_g + jnp.where(cond, suf[g + 1], jnp.int32(0))
    srev = jax.lax.rev(hv, dimensions=(0,))
    tot = plsc.cumsum(srev) + above_g
    kv = jnp.max(plsc.all_reduce_ffs(tot >= k))
    sel = jax.lax.iota(jnp.int32, 16) == kv
    hcnt = jnp.sum(jnp.where(sel, srev, zero16))
    tot_at = jnp.sum(jnp.where(sel, tot, zero16))
    return gsel * 16 + (15 - kv), tot_at - hcnt


def _sc_body(adj_hbm, noise_hbm, out_hbm, arow, nrow, vrow, hist):
    cid = jax.lax.axis_index("c")
    sid = jax.lax.axis_index("s")
    base = (sid * 2 + cid) * ROWS_PER_W
    zero16 = jnp.zeros((16,), jnp.int32)

    def row_body(r, _):
        row = base + r
        pltpu.sync_copy(adj_hbm.at[row], arow)
        pltpu.sync_copy(noise_hbm.at[row], nrow)

        for g in range(16):
            hist[pl.ds(g * 16, 16)] = zero16

        def p1(j, _):
            sl = pl.ds(j * 16, 16)
            v = arow[sl] + nrow[sl]
            vrow[sl] = v
            b = jnp.minimum((v * S1).astype(jnp.int32), HB - 1)
            cnt, last = plsc.scan_count(b)
            plsc.addupdate_scatter(hist, [b], cnt, mask=last)
            return 0

        jax.lax.fori_loop(0, NVREG, p1, 0)
        c1, ab1 = _sc_scan_hist(hist, jnp.int32(0))
        lo2 = c1.astype(jnp.float32) * W1

        for g in range(16):
            hist[pl.ds(g * 16, 16)] = zero16

        def p2(j, _):
            sl = pl.ds(j * 16, 16)
            v = vrow[sl]
            b1 = jnp.minimum((v * S1).astype(jnp.int32), HB - 1)
            m = b1 == c1
            b2 = jnp.clip(((v - lo2) * S2).astype(jnp.int32), 0, HB - 1)
            cnt, last = plsc.scan_count(b2, mask=m)
            plsc.addupdate_scatter(hist, [b2], cnt, mask=last)
            return 0

        jax.lax.fori_loop(0, NVREG, p2, 0)
        c2, ab2 = _sc_scan_hist(hist, ab1)
        lo3 = lo2 + c2.astype(jnp.float32) * W2

        for g in range(16):
            hist[pl.ds(g * 16, 16)] = zero16

        def p3(j, _):
            sl = pl.ds(j * 16, 16)
            v = vrow[sl]
            b1 = jnp.minimum((v * S1).astype(jnp.int32), HB - 1)
            b2 = jnp.clip(((v - lo2) * S2).astype(jnp.int32), 0, HB - 1)
            m = (b1 == c1) & (b2 == c2)
            b3 = jnp.clip(((v - lo3) * S3).astype(jnp.int32), 0, HB - 1)
            cnt, last = plsc.scan_count(b3, mask=m)
            plsc.addupdate_scatter(hist, [b3], cnt, mask=last)
            return 0

        jax.lax.fori_loop(0, NVREG, p3, 0)
        c3, _ab3 = _sc_scan_hist(hist, ab2)
        t = lo3 + (c3.astype(jnp.float32) - 2.0) * W3

        def p4(j, _):
            sl = pl.ds(j * 16, 16)
            nrow[sl] = jnp.where(vrow[sl] >= t, arow[sl], 0.0)
            return 0

        jax.lax.fori_loop(0, NVREG, p4, 0)
        pltpu.sync_copy(nrow, out_hbm.at[row])
        return 0

    jax.lax.fori_loop(0, ROWS_PER_W, row_body, 0)


_sc_select = functools.partial(
    pl.kernel,
    out_type=jax.ShapeDtypeStruct((N, N), jnp.float32),
    mesh=plsc.VectorSubcoreMesh(core_axis_name="c", subcore_axis_name="s"),
    scratch_types=[
        pltpu.VMEM((N,), jnp.float32),
        pltpu.VMEM((N,), jnp.float32),
        pltpu.VMEM((N,), jnp.float32),
        pltpu.VMEM((HB,), jnp.int32),
    ],
    compiler_params=pltpu.CompilerParams(needs_layout_passes=False),
)(_sc_body)


def kernel(inputs, inputs_init, outputs_init, idx, emb1_w, emb2_w):
    del idx, emb1_w, emb2_w
    z = jnp.squeeze(inputs, axis=1)
    z = jnp.transpose(z, (1, 0, 2)).reshape(N, C)
    zt = z.T
    a = _anomaly_vec(inputs_init, outputs_init)
    noise = _noise()

    ms = pl.pallas_call(
        _stats_body,
        out_shape=jax.ShapeDtypeStruct((2,), jnp.float32),
        out_specs=pl.BlockSpec(memory_space=pltpu.SMEM),
    )(z)

    adj = pl.pallas_call(
        _tc_body,
        grid=(N // BLK,),
        in_specs=[
            pl.BlockSpec((BLK, C), lambda i: (i, 0)),
            pl.BlockSpec((C, N), lambda i: (0, 0)),
            pl.BlockSpec((BLK, 1), lambda i: (i, 0)),
            pl.BlockSpec((1, N), lambda i: (0, 0)),
            pl.BlockSpec(memory_space=pltpu.SMEM),
        ],
        out_specs=pl.BlockSpec((BLK, N), lambda i: (i, 0)),
        out_shape=jax.ShapeDtypeStruct((N, N), jnp.float32),
    )(z, zt, a[:, None], a[None, :], ms)

    return _sc_select(adj, noise)


# speedup vs baseline: 1.5423x; 1.5423x over previous
import functools

import jax
import jax.numpy as jnp
from jax.experimental import pallas as pl
from jax.experimental.pallas import tpu as pltpu
from jax.experimental.pallas import tpu_sc as plsc

N = 4096
C = 768
K_TOPK = 32
NUM_SEL = 1229
BLK = 256

NW = 32
ROWS_PER_W = N // NW
NVREG = N // 16
HB = 256
RANGE = 3.02
W1 = RANGE / HB
W2 = W1 / HB
W3 = W2 / HB
S1 = HB / RANGE
S2 = HB / W1
S3 = HB / W2

_CONST_CACHE = {}


def _noise():
    if "noise" not in _CONST_CACHE:
        _CONST_CACHE["noise"] = (
            jax.random.uniform(jax.random.key(42), (N, N), dtype=jnp.float32) * 0.01
        )
    return _CONST_CACHE["noise"]


def _anomaly_vec(inputs_init, outputs_init):
    gap_list = jnp.mean(jnp.mean(jnp.abs(inputs_init - outputs_init), axis=1), axis=0)
    gap_list_ = jax.nn.sigmoid(jax.lax.stop_gradient(gap_list))
    neg_vals, small_idx = jax.lax.top_k(-gap_list_, NUM_SEL)
    topk_asc = -neg_vals
    topk_ = topk_asc[::-1]
    topk_idx = small_idx[::-1]
    threshold = jnp.mean(topk_)
    valid = topk_ > threshold
    anomaly_vals = jnp.where(valid, topk_, jnp.zeros_like(topk_))
    return jnp.zeros((N,), jnp.float32).at[topk_idx].set(anomaly_vals)


def _stats_body(z_ref, ms_ref):
    z = z_ref[...]
    g = jax.lax.dot_general(
        z, z, (((0,), (0,)), ((), ())), preferred_element_type=jnp.float32
    )
    sumsq = jnp.sum(g * g)
    s = jnp.sum(z, axis=0, keepdims=True)
    total = jnp.sum(s * s)
    n2 = float(N) * float(N)
    mean = total / n2
    var = (sumsq - total * (total / n2)) / (n2 - 1.0)
    ms_ref[0] = mean
    ms_ref[1] = jnp.sqrt(var)


def _tc_body(z_ref, zt_ref, arow_ref, acol_ref, ms_ref, out_ref):
    i = pl.program_id(0)
    base = i * BLK
    x = jax.lax.dot_general(
        z_ref[...], zt_ref[...], (((1,), (0,)), ((), ())),
        preferred_element_type=jnp.float32,
    )
    mean = ms_ref[0]
    std = ms_ref[1]
    adjb = jax.nn.sigmoid((x - mean) / (std + 1e-8))
    arow = arow_ref[...]
    acol = acol_ref[...]
    adjb = adjb + arow + acol
    col_ids = jax.lax.broadcasted_iota(jnp.int32, (BLK, N), 1)
    row_ids = jax.lax.broadcasted_iota(jnp.int32, (BLK, N), 0) + base
    out_ref[...] = adjb - jnp.where(col_ids == row_ids, arow, 0.0)


def _sc_scan_hist(hist, above):
    gs = [hist[pl.ds(g * 16, 16)] for g in range(16)]
    sums = [jnp.sum(g) for g in gs]
    suf = [None] * 17
    acc = above
    suf[16] = above
    for g in range(15, -1, -1):
        acc = acc + sums[g]
        suf[g] = acc
    k = jnp.int32(K_TOPK)
    zero16 = jnp.zeros((16,), jnp.int32)
    gsel = jnp.int32(0)
    hv = zero16
    above_g = jnp.int32(0)
    for g in range(16):
        cond = (suf[g] >= k) & (suf[g + 1] < k)
        gsel = gsel + jnp.where(cond, jnp.int32(g), jnp.int32(0))
        hv = hv + jnp.where(cond, gs[g], zero16)
        above_g = above_g + jnp.where(cond, suf[g + 1], jnp.int32(0))
    srev = jax.lax.rev(hv, dimensions=(0,))
    tot = plsc.cumsum(srev) + above_g
    kv = jnp.max(plsc.all_reduce_ffs(tot >= k))
    sel = jax.lax.iota(jnp.int32, 16) == kv
    hcnt = jnp.sum(jnp.where(sel, srev, zero16))
    tot_at = jnp.sum(jnp.where(sel, tot, zero16))
    return gsel * 16 + (15 - kv), tot_at - hcnt


def _sc_body(adj_hbm, noise_hbm, out_hbm, arow, nrow, vrow, obuf, cand, hist,
             sem_out):
    cid = jax.lax.axis_index("c")
    sid = jax.lax.axis_index("s")
    base = (sid * 2 + cid) * ROWS_PER_W
    zero16 = jnp.zeros((16,), jnp.int32)
    lane = jax.lax.iota(jnp.int32, 16)

    def row_body(r, _):
        row = base + r
        pltpu.sync_copy(adj_hbm.at[row], arow)
        pltpu.sync_copy(noise_hbm.at[row], nrow)

        for g in range(16):
            hist[pl.ds(g * 16, 16)] = zero16

        def p1(jj, _):
            for u in range(8):
                sl = pl.ds((jj * 8 + u) * 16, 16)
                v = arow[sl] + nrow[sl]
                vrow[sl] = v
                b = jnp.minimum((v * S1).astype(jnp.int32), HB - 1)
                cnt, last = plsc.scan_count(b)
                plsc.addupdate_scatter(hist, [b], cnt, mask=last)
            return 0

        jax.lax.fori_loop(0, NVREG // 8, p1, 0)
        c1, ab1 = _sc_scan_hist(hist, jnp.int32(0))
        rneed = jnp.int32(K_TOPK) - ab1
        c1f = c1.astype(jnp.float32)

        def p2(jj, off):
            for u in range(4):
                sl = pl.ds((jj * 4 + u) * 16, 16)
                v = vrow[sl]
                b1 = jnp.minimum((v * S1).astype(jnp.int32), HB - 1)
                m = b1 == c1
                plsc.store_compressed(cand.at[pl.ds(off, 16)], v, mask=m)
                off = off + jnp.max(plsc.all_reduce_population_count(m))
            return off

        mcnt = jax.lax.fori_loop(0, NVREG // 4, p2, jnp.int32(0))
        nv = jax.lax.shift_right_logical(mcnt + 15, 4)

        lo0 = c1f * W1 * (1.0 - 1e-6) - 1e-9
        hi0 = (c1f + 1.0) * W1 * (1.0 + 1e-6) + 1e-9

        def bis(_, carry):
            lo, hi = carry
            mid = (lo + hi) * 0.5

            def cb(j, acc):
                c = cand[pl.ds(j * 16, 16)]
                valid = (lane + j * 16) < mcnt
                return acc + jnp.sum(jnp.where(valid & (c >= mid), 1, 0))

            cnt = jax.lax.fori_loop(0, nv, cb, jnp.int32(0))
            pred = cnt >= rneed
            return (jnp.where(pred, mid, lo), jnp.where(pred, hi, mid))

        t, _hi = jax.lax.fori_loop(0, 28, bis, (lo0, hi0))

        @pl.when(r > 0)
        def _():
            pltpu.make_async_copy(obuf, out_hbm.at[row - 1], sem_out).wait()

        def p4(jj, _):
            for u in range(8):
                sl = pl.ds((jj * 8 + u) * 16, 16)
                obuf[sl] = jnp.where(vrow[sl] >= t, arow[sl], 0.0)
            return 0

        jax.lax.fori_loop(0, NVREG // 8, p4, 0)
        pltpu.async_copy(obuf, out_hbm.at[row], sem_out)
        return 0

    jax.lax.fori_loop(0, ROWS_PER_W, row_body, 0)
    pltpu.make_async_copy(obuf, out_hbm.at[base + ROWS_PER_W - 1], sem_out).wait()


_sc_select = functools.partial(
    pl.kernel,
    out_type=jax.ShapeDtypeStruct((N, N), jnp.float32),
    mesh=plsc.VectorSubcoreMesh(core_axis_name="c", subcore_axis_name="s"),
    scratch_types=[
        pltpu.VMEM((N,), jnp.float32),
        pltpu.VMEM((N,), jnp.float32),
        pltpu.VMEM((N,), jnp.float32),
        pltpu.VMEM((N,), jnp.float32),
        pltpu.VMEM((N + 16,), jnp.float32),
        pltpu.VMEM((HB,), jnp.int32),
        pltpu.SemaphoreType.DMA,
    ],
    compiler_params=pltpu.CompilerParams(needs_layout_passes=False),
)(_sc_body)


def kernel(inputs, inputs_init, outputs_init, idx, emb1_w, emb2_w):
    del idx, emb1_w, emb2_w
    z = jnp.squeeze(inputs, axis=1)
    z = jnp.transpose(z, (1, 0, 2)).reshape(N, C)
    zt = z.T
    a = _anomaly_vec(inputs_init, outputs_init)
    noise = _noise()

    ms = pl.pallas_call(
        _stats_body,
        out_shape=jax.ShapeDtypeStruct((2,), jnp.float32),
        out_specs=pl.BlockSpec(memory_space=pltpu.SMEM),
    )(z)

    adj = pl.pallas_call(
        _tc_body,
        grid=(N // BLK,),
        in_specs=[
            pl.BlockSpec((BLK, C), lambda i: (i, 0)),
            pl.BlockSpec((C, N), lambda i: (0, 0)),
            pl.BlockSpec((BLK, 1), lambda i: (i, 0)),
            pl.BlockSpec((1, N), lambda i: (0, 0)),
            pl.BlockSpec(memory_space=pltpu.SMEM),
        ],
        out_specs=pl.BlockSpec((BLK, N), lambda i: (i, 0)),
        out_shape=jax.ShapeDtypeStruct((N, N), jnp.float32),
    )(z, zt, a[:, None], a[None, :], ms)

    return _sc_select(adj, noise)


# speedup vs baseline: 1.5875x; 1.0293x over previous
import functools

import jax
import jax.numpy as jnp
from jax.experimental import pallas as pl
from jax.experimental.pallas import tpu as pltpu
from jax.experimental.pallas import tpu_sc as plsc

N = 4096
C = 768
K_TOPK = 32
NUM_SEL = 1229
BLK = 256

NW = 32
ROWS_PER_W = N // NW
NVREG = N // 16
HB = 256
NSUB = 8
RANGE = 3.02
W1 = RANGE / HB
W2 = W1 / HB
W3 = W2 / HB
S1 = HB / RANGE
S2 = HB / W1
S3 = HB / W2

_CONST_CACHE = {}


def _noise():
    if "noise" not in _CONST_CACHE:
        _CONST_CACHE["noise"] = (
            jax.random.uniform(jax.random.key(42), (N, N), dtype=jnp.float32) * 0.01
        )
    return _CONST_CACHE["noise"]


def _anomaly_vec(inputs_init, outputs_init):
    gap_list = jnp.mean(jnp.mean(jnp.abs(inputs_init - outputs_init), axis=1), axis=0)
    gap_list_ = jax.nn.sigmoid(jax.lax.stop_gradient(gap_list))
    neg_vals, small_idx = jax.lax.top_k(-gap_list_, NUM_SEL)
    topk_asc = -neg_vals
    topk_ = topk_asc[::-1]
    topk_idx = small_idx[::-1]
    threshold = jnp.mean(topk_)
    valid = topk_ > threshold
    anomaly_vals = jnp.where(valid, topk_, jnp.zeros_like(topk_))
    return jnp.zeros((N,), jnp.float32).at[topk_idx].set(anomaly_vals)


def _stats_body(z_ref, ms_ref):
    z = z_ref[...]
    g = jax.lax.dot_general(
        z, z, (((0,), (0,)), ((), ())), preferred_element_type=jnp.float32
    )
    sumsq = jnp.sum(g * g)
    s = jnp.sum(z, axis=0, keepdims=True)
    total = jnp.sum(s * s)
    n2 = float(N) * float(N)
    mean = total / n2
    var = (sumsq - total * (total / n2)) / (n2 - 1.0)
    ms_ref[0] = mean
    ms_ref[1] = jnp.sqrt(var)


def _tc_body(z_ref, zt_ref, arow_ref, acol_ref, ms_ref, out_ref):
    i = pl.program_id(0)
    base = i * BLK
    x = jax.lax.dot_general(
        z_ref[...], zt_ref[...], (((1,), (0,)), ((), ())),
        preferred_element_type=jnp.float32,
    )
    mean = ms_ref[0]
    std = ms_ref[1]
    adjb = jax.nn.sigmoid((x - mean) / (std + 1e-8))
    arow = arow_ref[...]
    acol = acol_ref[...]
    adjb = adjb + arow + acol
    col_ids = jax.lax.broadcasted_iota(jnp.int32, (BLK, N), 1)
    row_ids = jax.lax.broadcasted_iota(jnp.int32, (BLK, N), 0) + base
    out_ref[...] = adjb - jnp.where(col_ids == row_ids, arow, 0.0)


def _sc_scan_hist(hist, above):
    gs = []
    for g in range(16):
        acc = hist[pl.ds(g * 16, 16)]
        for h in range(1, NSUB):
            acc = acc + hist[pl.ds(h * HB + g * 16, 16)]
        gs.append(acc)
    sums = [jnp.sum(g) for g in gs]
    suf = [None] * 17
    acc = above
    suf[16] = above
    for g in range(15, -1, -1):
        acc = acc + sums[g]
        suf[g] = acc
    k = jnp.int32(K_TOPK)
    zero16 = jnp.zeros((16,), jnp.int32)
    gsel = jnp.int32(0)
    hv = zero16
    above_g = jnp.int32(0)
    for g in range(16):
        cond = (suf[g] >= k) & (suf[g + 1] < k)
        gsel = gsel + jnp.where(cond, jnp.int32(g), jnp.int32(0))
        hv = hv + jnp.where(cond, gs[g], zero16)
        above_g = above_g + jnp.where(cond, suf[g + 1], jnp.int32(0))
    srev = jax.lax.rev(hv, dimensions=(0,))
    tot = plsc.cumsum(srev) + above_g
    kv = jnp.max(plsc.all_reduce_ffs(tot >= k))
    sel = jax.lax.iota(jnp.int32, 16) == kv
    hcnt = jnp.sum(jnp.where(sel, srev, zero16))
    tot_at = jnp.sum(jnp.where(sel, tot, zero16))
    return gsel * 16 + (15 - kv), tot_at - hcnt


def _sc_body(adj_hbm, noise_hbm, out_hbm, arow, nrow, vrow, obuf, cand, hist,
             sem_out):
    cid = jax.lax.axis_index("c")
    sid = jax.lax.axis_index("s")
    base = (sid * 2 + cid) * ROWS_PER_W
    zero16 = jnp.zeros((16,), jnp.int32)
    lane = jax.lax.iota(jnp.int32, 16)

    def row_body(r, _):
        row = base + r
        pltpu.sync_copy(adj_hbm.at[row], arow)
        pltpu.sync_copy(noise_hbm.at[row], nrow)

        for h in range(NSUB):
            for g in range(16):
                hist[pl.ds(h * HB + g * 16, 16)] = zero16

        def p1(jj, _):
            for u in range(NSUB):
                sl = pl.ds((jj * NSUB + u) * 16, 16)
                v = arow[sl] + nrow[sl]
                vrow[sl] = v
                b = jnp.minimum((v * S1).astype(jnp.int32), HB - 1)
                cnt, last = plsc.scan_count(b)
                plsc.addupdate_scatter(hist, [b + u * HB], cnt, mask=last)
            return 0

        jax.lax.fori_loop(0, NVREG // NSUB, p1, 0)
        c1, ab1 = _sc_scan_hist(hist, jnp.int32(0))
        rneed = jnp.int32(K_TOPK) - ab1
        c1f = c1.astype(jnp.float32)

        def p2(jj, off):
            for u in range(4):
                sl = pl.ds((jj * 4 + u) * 16, 16)
                v = vrow[sl]
                b1 = jnp.minimum((v * S1).astype(jnp.int32), HB - 1)
                m = b1 == c1
                plsc.store_compressed(cand.at[pl.ds(off, 16)], v, mask=m)
                pc = plsc.all_reduce_population_count(m)
                off = off + jnp.squeeze(jax.lax.slice(pc, (0,), (1,)))
            return off

        mcnt = jax.lax.fori_loop(0, NVREG // 4, p2, jnp.int32(0))
        nv = jax.lax.shift_right_logical(mcnt + 15, 4)

        lo0 = c1f * W1 * (1.0 - 1e-6) - 1e-9
        hi0 = (c1f + 1.0) * W1 * (1.0 + 1e-6) + 1e-9

        def bis(_, carry):
            lo, hi = carry
            mid = (lo + hi) * 0.5

            def cb(j, acc):
                c = cand[pl.ds(j * 16, 16)]
                valid = (lane + j * 16) < mcnt
                return acc + jnp.where(valid & (c >= mid), 1, 0)

            accv = jax.lax.fori_loop(0, nv, cb, zero16)
            pred = jnp.sum(accv) >= rneed
            return (jnp.where(pred, mid, lo), jnp.where(pred, hi, mid))

        t, _hi = jax.lax.fori_loop(0, 28, bis, (lo0, hi0))

        @pl.when(r > 0)
        def _():
            pltpu.make_async_copy(obuf, out_hbm.at[row - 1], sem_out).wait()

        def p4(jj, _):
            for u in range(8):
                sl = pl.ds((jj * 8 + u) * 16, 16)
                obuf[sl] = jnp.where(vrow[sl] >= t, arow[sl], 0.0)
            return 0

        jax.lax.fori_loop(0, NVREG // 8, p4, 0)
        pltpu.async_copy(obuf, out_hbm.at[row], sem_out)
        return 0

    jax.lax.fori_loop(0, ROWS_PER_W, row_body, 0)
    pltpu.make_async_copy(obuf, out_hbm.at[base + ROWS_PER_W - 1], sem_out).wait()


_sc_select = functools.partial(
    pl.kernel,
    out_type=jax.ShapeDtypeStruct((N, N), jnp.float32),
    mesh=plsc.VectorSubcoreMesh(core_axis_name="c", subcore_axis_name="s"),
    scratch_types=[
        pltpu.VMEM((N,), jnp.float32),
        pltpu.VMEM((N,), jnp.float32),
        pltpu.VMEM((N,), jnp.float32),
        pltpu.VMEM((N,), jnp.float32),
        pltpu.VMEM((N + 16,), jnp.float32),
        pltpu.VMEM((NSUB * HB,), jnp.int32),
        pltpu.SemaphoreType.DMA,
    ],
    compiler_params=pltpu.CompilerParams(needs_layout_passes=False),
)(_sc_body)


def kernel(inputs, inputs_init, outputs_init, idx, emb1_w, emb2_w):
    del idx, emb1_w, emb2_w
    z = jnp.squeeze(inputs, axis=1)
    z = jnp.transpose(z, (1, 0, 2)).reshape(N, C)
    zt = z.T
    a = _anomaly_vec(inputs_init, outputs_init)
    noise = _noise()

    ms = pl.pallas_call(
        _stats_body,
        out_shape=jax.ShapeDtypeStruct((2,), jnp.float32),
        out_specs=pl.BlockSpec(memory_space=pltpu.SMEM),
    )(z)

    adj = pl.pallas_call(
        _tc_body,
        grid=(N // BLK,),
        in_specs=[
            pl.BlockSpec((BLK, C), lambda i: (i, 0)),
            pl.BlockSpec((C, N), lambda i: (0, 0)),
            pl.BlockSpec((BLK, 1), lambda i: (i, 0)),
            pl.BlockSpec((1, N), lambda i: (0, 0)),
            pl.BlockSpec(memory_space=pltpu.SMEM),
        ],
        out_specs=pl.BlockSpec((BLK, N), lambda i: (i, 0)),
        out_shape=jax.ShapeDtypeStruct((N, N), jnp.float32),
    )(z, zt, a[:, None], a[None, :], ms)

    return _sc_select(adj, noise)
